# baseline (speedup 1.0000x reference)
import functools

import jax
import jax.numpy as jnp
from jax import lax
from jax.experimental import pallas as pl
from jax.experimental.pallas import tpu as pltpu
from jax.experimental.pallas import tpu_sc as plsc

_NC = 2
_NS = 16
_NW = _NC * _NS


_CH = 80


def _sc_degree(e4, zeros_n):
    n = zeros_n.shape[0]
    nch, ch = e4.shape[2:]
    mesh = plsc.VectorSubcoreMesh(core_axis_name="c", subcore_axis_name="s")

    @functools.partial(
        pl.kernel,
        out_type=jax.ShapeDtypeStruct((_NW, n), jnp.float32),
        mesh=mesh,
        scratch_types=[
            pltpu.VMEM((nch, ch), jnp.int32),
            pltpu.VMEM((n,), jnp.float32),
        ],
        compiler_params=pltpu.CompilerParams(needs_layout_passes=False,
                                             use_tc_tiling_on_sc=False),
    )
    def k(e_hbm, z_hbm, out_hbm, dst_all, hist):
        c = lax.axis_index("c")
        s = lax.axis_index("s")
        wid = c * _NS + s
        pltpu.sync_copy(e_hbm.at[1, wid], dst_all)
        pltpu.sync_copy(z_hbm, hist)
        ones = jnp.full((16,), 1.0, jnp.float32)

        def body(i, carry):
            for j in range(ch // 16):
                idx = dst_all[i, pl.ds(j * 16, 16)]
                plsc.addupdate_scatter(hist, [idx], ones)
            return carry

        lax.fori_loop(0, nch, body, 0)
        pltpu.sync_copy(hist, out_hbm.at[wid])

    return k(e4, zeros_n)


def _sc_scatter(g, e4, zeros_nf):
    n, f = g.shape
    nch, ch = e4.shape[2:]
    npair = (nch - 1) // 2
    rpt = n // _NS
    mesh = plsc.VectorSubcoreMesh(core_axis_name="c", subcore_axis_name="s")

    @functools.partial(
        pl.kernel,
        out_type=jax.ShapeDtypeStruct((_NC, n, f), jnp.float32),
        mesh=mesh,
        scratch_types=[
            pltpu.VMEM((nch, ch), jnp.int32),
            pltpu.VMEM((nch, ch), jnp.int32),
            pltpu.VMEM((ch, f), jnp.float32),
            pltpu.VMEM((ch, f), jnp.float32),
            pltpu.VMEM_SHARED((n, f), jnp.float32),
            pltpu.SemaphoreType.DMA,
            pltpu.SemaphoreType.DMA,
        ],
        compiler_params=pltpu.CompilerParams(needs_layout_passes=False,
                                             use_tc_tiling_on_sc=False),
    )
    def k(g_hbm, e_hbm, z_hbm, out_hbm,
          src_all, dst_all, rows0, rows1, acc_sh, sem0, sem1):
        c = lax.axis_index("c")
        s = lax.axis_index("s")
        wid = c * _NS + s
        r0 = s * rpt
        pltpu.sync_copy(e_hbm.at[0, wid], src_all)
        pltpu.sync_copy(e_hbm.at[1, wid], dst_all)
        pltpu.sync_copy(z_hbm.at[pl.ds(r0, rpt)], acc_sh.at[pl.ds(r0, rpt)])
        plsc.subcore_barrier()

        pltpu.async_copy(g_hbm.at[src_all.at[0]], rows0, sem0)

        def pair(j, carry):
            b1 = 2 * j + 1
            cp1 = pltpu.async_copy(g_hbm.at[src_all.at[b1]], rows1, sem1)
            pltpu.make_async_copy(g_hbm.at[src_all.at[b1 - 1]], rows0,
                                  sem0).wait()
            pltpu.sync_copy(rows0, acc_sh.at[dst_all.at[b1 - 1]], add=True)
            pltpu.async_copy(g_hbm.at[src_all.at[b1 + 1]], rows0, sem0)
            cp1.wait()
            pltpu.sync_copy(rows1, acc_sh.at[dst_all.at[b1]], add=True)
            return carry

        lax.fori_loop(0, npair, pair, 0)
        pltpu.make_async_copy(g_hbm.at[src_all.at[nch - 1]], rows0, sem0).wait()
        pltpu.sync_copy(rows0, acc_sh.at[dst_all.at[nch - 1]], add=True)

        plsc.subcore_barrier()
        pltpu.sync_copy(acc_sh.at[pl.ds(r0, rpt)],
                        out_hbm.at[c, pl.ds(r0, rpt)])

    return k(g, e4, zeros_nf)


def _tc_h(x, w1):
    n = x.shape[0]
    f = w1.shape[1]

    def body(x_ref, w_ref, o_ref):
        o_ref[...] = jnp.dot(x_ref[...], w_ref[...],
                             preferred_element_type=jnp.float32)

    return pl.pallas_call(
        body,
        out_shape=jax.ShapeDtypeStruct((n, f), jnp.float32),
    )(x, w1)


def _tc_dinv_g1(hist, h):
    n, f = h.shape

    def body(hi_ref, h_ref, d_ref, g_ref):
        deg = 1.0 + jnp.sum(hi_ref[...], axis=0, keepdims=True)
        dcol = lax.rsqrt(deg).reshape(n, 1)
        d_ref[...] = dcol
        g_ref[...] = h_ref[...] * dcol

    return pl.pallas_call(
        body,
        out_shape=(jax.ShapeDtypeStruct((n, 1), jnp.float32),
                   jax.ShapeDtypeStruct((n, f), jnp.float32)),
    )(hist, h)


def _tc_g2(s1, g1, dinv_col, b1):
    n, f = g1.shape

    def body(s_ref, g_ref, d_ref, b_ref, o_ref):
        d = d_ref[...]
        m = d * (s_ref[0] + s_ref[1] + g_ref[...]) + b_ref[...]
        o_ref[...] = jnp.maximum(m, 0.0) * d

    return pl.pallas_call(
        body,
        out_shape=jax.ShapeDtypeStruct((n, f), jnp.float32),
    )(s1, g1, dinv_col, b1.reshape(1, f))


def _tc_final(s2, g2, dinv_col, w2, b2, k0, k1, k2, fc_wt, fc_b):
    n = g2.shape[0]
    ncls = fc_wt.shape[1]

    def body(s_ref, g_ref, d_ref, w2_ref, b2_ref, k0_ref, k1_ref, k2_ref,
             fw_ref, fb_ref, o_ref):
        m = d_ref[...] * (s_ref[0] + s_ref[1] + g_ref[...])
        h2 = jnp.dot(m, w2_ref[...],
                     preferred_element_type=jnp.float32) + b2_ref[...]
        a = jnp.dot(h2, k0_ref[...], preferred_element_type=jnp.float32)
        b = jnp.dot(h2, k1_ref[...], preferred_element_type=jnp.float32)
        c = jnp.dot(h2, k2_ref[...], preferred_element_type=jnp.float32)
        row = lax.broadcasted_iota(jnp.int32, a.shape, 0)
        a_dn = jnp.where(row == 0, 0.0, pltpu.roll(a, 1, 0))
        c_up = jnp.where(row == n - 1, 0.0, pltpu.roll(c, n - 1, 0))
        y = b + a_dn + c_up
        p = jnp.max(y, axis=0, keepdims=True)
        o_ref[...] = jnp.dot(p, fw_ref[...],
                             preferred_element_type=jnp.float32) + fb_ref[...]

    return pl.pallas_call(
        body,
        out_shape=jax.ShapeDtypeStruct((1, ncls), jnp.float32),
    )(s2, g2, dinv_col, w2, b2.reshape(1, -1), k0, k1, k2,
      fc_wt, fc_b.reshape(1, -1))


def kernel(x, edge_index, W1, b1, W2, b2, conv_w, conv_b, fc_w, fc_b):
    n = x.shape[0]
    f = W1.shape[1]
    e = edge_index.shape[1]
    ept = e // _NW
    nch = ept // _CH
    assert ept % _CH == 0 and nch % 2 == 1
    e4 = edge_index.reshape(2, _NW, nch, _CH)

    zeros_n = jnp.zeros((n,), jnp.float32)
    zeros_nf = jnp.zeros((n, f), jnp.float32)

    h = _tc_h(x, W1)
    hist = _sc_degree(e4, zeros_n)
    dinv_col, g1 = _tc_dinv_g1(hist, h)
    s1 = _sc_scatter(g1, e4, zeros_nf)
    g2 = _tc_g2(s1, g1, dinv_col, b1)
    s2 = _sc_scatter(g2, e4, zeros_nf)

    k0 = conv_w[:, :, 0, 0].T
    k1 = conv_w[:, :, 0, 1].T
    k2 = conv_w[:, :, 0, 2].T
    y_bias = conv_b.reshape(1, -1)

    out = _tc_final(s2, g2, dinv_col, W2, b2, k0, k1, k2,
                    fc_w.T, fc_b + y_bias @ fc_w.T)
    return out


# speedup vs baseline: 1.4358x; 1.4358x over previous
import functools

import jax
import jax.numpy as jnp
from jax import lax
from jax.experimental import pallas as pl
from jax.experimental.pallas import tpu as pltpu
from jax.experimental.pallas import tpu_sc as plsc

_NC = 2
_NS = 16
_NW = _NC * _NS


_CH = 125


def _sc_degree(e4, zeros_n):
    n = zeros_n.shape[0]
    nch, ch = e4.shape[2:]
    mesh = plsc.VectorSubcoreMesh(core_axis_name="c", subcore_axis_name="s")

    @functools.partial(
        pl.kernel,
        out_type=jax.ShapeDtypeStruct((_NW, n), jnp.float32),
        mesh=mesh,
        scratch_types=[
            pltpu.VMEM((nch, ch), jnp.int32),
            pltpu.VMEM((n,), jnp.float32),
        ],
        compiler_params=pltpu.CompilerParams(needs_layout_passes=False,
                                             use_tc_tiling_on_sc=False),
    )
    def k(e_hbm, z_hbm, out_hbm, dst_all, hist):
        c = lax.axis_index("c")
        s = lax.axis_index("s")
        wid = c * _NS + s
        pltpu.sync_copy(e_hbm.at[1, wid], dst_all)
        pltpu.sync_copy(z_hbm, hist)
        ones = jnp.full((16,), 1.0, jnp.float32)

        def body(i, carry):
            for j in range(ch // 16):
                idx = dst_all[i, pl.ds(j * 16, 16)]
                plsc.addupdate_scatter(hist, [idx], ones)
            return carry

        lax.fori_loop(0, nch, body, 0)
        pltpu.sync_copy(hist, out_hbm.at[wid])

    return k(e4, zeros_n)


def _sc_scatter(g, e4, zeros_nf):
    n, f = g.shape
    nch, ch = e4.shape[2:]
    npair = (nch - 1) // 2
    rpt = n // _NS
    mesh = plsc.VectorSubcoreMesh(core_axis_name="c", subcore_axis_name="s")

    @functools.partial(
        pl.kernel,
        out_type=jax.ShapeDtypeStruct((_NC, n, f), jnp.float32),
        mesh=mesh,
        scratch_types=[
            pltpu.VMEM((nch, ch), jnp.int32),
            pltpu.VMEM((nch, ch), jnp.int32),
            pltpu.VMEM((ch, f), jnp.float32),
            pltpu.VMEM((ch, f), jnp.float32),
            pltpu.VMEM_SHARED((n, f), jnp.float32),
            pltpu.VMEM_SHARED((n, f), jnp.float32),
            pltpu.SemaphoreType.DMA,
            pltpu.SemaphoreType.DMA,
        ],
        compiler_params=pltpu.CompilerParams(needs_layout_passes=False,
                                             use_tc_tiling_on_sc=False),
    )
    def k(g_hbm, e_hbm, z_hbm, out_hbm,
          src_all, dst_all, rows0, rows1, g_sh, acc_sh, sem0, sem1):
        c = lax.axis_index("c")
        s = lax.axis_index("s")
        wid = c * _NS + s
        r0 = s * rpt
        pltpu.sync_copy(e_hbm.at[0, wid], src_all)
        pltpu.sync_copy(e_hbm.at[1, wid], dst_all)
        pltpu.sync_copy(g_hbm.at[pl.ds(r0, rpt)], g_sh.at[pl.ds(r0, rpt)])
        pltpu.sync_copy(z_hbm.at[pl.ds(r0, rpt)], acc_sh.at[pl.ds(r0, rpt)])
        plsc.subcore_barrier()

        pltpu.async_copy(g_sh.at[src_all.at[0]], rows0, sem0)

        def pair(j, carry):
            b1 = 2 * j + 1
            cp1 = pltpu.async_copy(g_sh.at[src_all.at[b1]], rows1, sem1)
            pltpu.make_async_copy(g_sh.at[src_all.at[b1 - 1]], rows0,
                                  sem0).wait()
            pltpu.sync_copy(rows0, acc_sh.at[dst_all.at[b1 - 1]], add=True)
            pltpu.async_copy(g_sh.at[src_all.at[b1 + 1]], rows0, sem0)
            cp1.wait()
            pltpu.sync_copy(rows1, acc_sh.at[dst_all.at[b1]], add=True)
            return carry

        lax.fori_loop(0, npair, pair, 0)
        if nch % 2 == 1:
            pltpu.make_async_copy(g_sh.at[src_all.at[nch - 1]], rows0,
                                  sem0).wait()
            pltpu.sync_copy(rows0, acc_sh.at[dst_all.at[nch - 1]], add=True)
        else:
            cp1 = pltpu.async_copy(g_sh.at[src_all.at[nch - 1]], rows1, sem1)
            pltpu.make_async_copy(g_sh.at[src_all.at[nch - 2]], rows0,
                                  sem0).wait()
            pltpu.sync_copy(rows0, acc_sh.at[dst_all.at[nch - 2]], add=True)
            cp1.wait()
            pltpu.sync_copy(rows1, acc_sh.at[dst_all.at[nch - 1]], add=True)

        plsc.subcore_barrier()
        pltpu.sync_copy(acc_sh.at[pl.ds(r0, rpt)],
                        out_hbm.at[c, pl.ds(r0, rpt)])

    return k(g, e4, zeros_nf)


def _tc_h(x, w1):
    n = x.shape[0]
    f = w1.shape[1]

    def body(x_ref, w_ref, o_ref):
        o_ref[...] = jnp.dot(x_ref[...], w_ref[...],
                             preferred_element_type=jnp.float32)

    return pl.pallas_call(
        body,
        out_shape=jax.ShapeDtypeStruct((n, f), jnp.float32),
    )(x, w1)


def _tc_dinv_g1(hist, h):
    n, f = h.shape

    def body(hi_ref, h_ref, d_ref, g_ref):
        deg = 1.0 + jnp.sum(hi_ref[...], axis=0, keepdims=True)
        dcol = lax.rsqrt(deg).reshape(n, 1)
        d_ref[...] = dcol
        g_ref[...] = h_ref[...] * dcol

    return pl.pallas_call(
        body,
        out_shape=(jax.ShapeDtypeStruct((n, 1), jnp.float32),
                   jax.ShapeDtypeStruct((n, f), jnp.float32)),
    )(hist, h)


def _tc_g2(s1, g1, dinv_col, b1):
    n, f = g1.shape

    def body(s_ref, g_ref, d_ref, b_ref, o_ref):
        d = d_ref[...]
        m = d * (s_ref[0] + s_ref[1] + g_ref[...]) + b_ref[...]
        o_ref[...] = jnp.maximum(m, 0.0) * d

    return pl.pallas_call(
        body,
        out_shape=jax.ShapeDtypeStruct((n, f), jnp.float32),
    )(s1, g1, dinv_col, b1.reshape(1, f))


def _tc_final(s2, g2, dinv_col, w2, b2, k0, k1, k2, fc_wt, fc_b):
    n = g2.shape[0]
    ncls = fc_wt.shape[1]

    def body(s_ref, g_ref, d_ref, w2_ref, b2_ref, k0_ref, k1_ref, k2_ref,
             fw_ref, fb_ref, o_ref):
        m = d_ref[...] * (s_ref[0] + s_ref[1] + g_ref[...])
        h2 = jnp.dot(m, w2_ref[...],
                     preferred_element_type=jnp.float32) + b2_ref[...]
        a = jnp.dot(h2, k0_ref[...], preferred_element_type=jnp.float32)
        b = jnp.dot(h2, k1_ref[...], preferred_element_type=jnp.float32)
        c = jnp.dot(h2, k2_ref[...], preferred_element_type=jnp.float32)
        row = lax.broadcasted_iota(jnp.int32, a.shape, 0)
        a_dn = jnp.where(row == 0, 0.0, pltpu.roll(a, 1, 0))
        c_up = jnp.where(row == n - 1, 0.0, pltpu.roll(c, n - 1, 0))
        y = b + a_dn + c_up
        p = jnp.max(y, axis=0, keepdims=True)
        o_ref[...] = jnp.dot(p, fw_ref[...],
                             preferred_element_type=jnp.float32) + fb_ref[...]

    return pl.pallas_call(
        body,
        out_shape=jax.ShapeDtypeStruct((1, ncls), jnp.float32),
    )(s2, g2, dinv_col, w2, b2.reshape(1, -1), k0, k1, k2,
      fc_wt, fc_b.reshape(1, -1))


def kernel(x, edge_index, W1, b1, W2, b2, conv_w, conv_b, fc_w, fc_b):
    n = x.shape[0]
    f = W1.shape[1]
    e = edge_index.shape[1]
    ept = e // _NW
    nch = ept // _CH
    assert ept % _CH == 0
    e4 = edge_index.reshape(2, _NW, nch, _CH)

    zeros_n = jnp.zeros((n,), jnp.float32)
    zeros_nf = jnp.zeros((n, f), jnp.float32)

    h = _tc_h(x, W1)
    hist = _sc_degree(e4, zeros_n)
    dinv_col, g1 = _tc_dinv_g1(hist, h)
    s1 = _sc_scatter(g1, e4, zeros_nf)
    g2 = _tc_g2(s1, g1, dinv_col, b1)
    s2 = _sc_scatter(g2, e4, zeros_nf)

    k0 = conv_w[:, :, 0, 0].T
    k1 = conv_w[:, :, 0, 1].T
    k2 = conv_w[:, :, 0, 2].T
    y_bias = conv_b.reshape(1, -1)

    out = _tc_final(s2, g2, dinv_col, W2, b2, k0, k1, k2,
                    fc_w.T, fc_b + y_bias @ fc_w.T)
    return out


# speedup vs baseline: 1.4461x; 1.0072x over previous
import functools

import jax
import jax.numpy as jnp
from jax import lax
from jax.experimental import pallas as pl
from jax.experimental.pallas import tpu as pltpu
from jax.experimental.pallas import tpu_sc as plsc

_NC = 2
_NS = 16
_NW = _NC * _NS
_L = 16

_CH = 80

_SC_PARAMS = pltpu.CompilerParams(needs_layout_passes=False,
                                  use_tc_tiling_on_sc=False)


def _mesh():
    return plsc.VectorSubcoreMesh(core_axis_name="c", subcore_axis_name="s")


def _rsqrt16(x):
    y = plsc.bitcast(x, jnp.int32)
    y = jnp.int32(0x5F3759DF) - lax.shift_right_logical(y, 1)
    r = plsc.bitcast(y, jnp.float32)
    for _ in range(3):
        r = r * (1.5 - 0.5 * x * r * r)
    return r


def _edge_loop(g_sh, acc_sh, src_all, dst_all, rows0, rows1, sem0, sem1,
               nch):
    npair = (nch - 1) // 2
    pltpu.async_copy(g_sh.at[src_all.at[0]], rows0, sem0)

    def pair(j, carry):
        b1 = 2 * j + 1
        cp1 = pltpu.async_copy(g_sh.at[src_all.at[b1]], rows1, sem1)
        pltpu.make_async_copy(g_sh.at[src_all.at[b1 - 1]], rows0,
                              sem0).wait()
        pltpu.sync_copy(rows0, acc_sh.at[dst_all.at[b1 - 1]], add=True)
        pltpu.async_copy(g_sh.at[src_all.at[b1 + 1]], rows0, sem0)
        cp1.wait()
        pltpu.sync_copy(rows1, acc_sh.at[dst_all.at[b1]], add=True)
        return carry

    lax.fori_loop(0, npair, pair, 0)
    if nch % 2 == 1:
        pltpu.make_async_copy(g_sh.at[src_all.at[nch - 1]], rows0,
                              sem0).wait()
        pltpu.sync_copy(rows0, acc_sh.at[dst_all.at[nch - 1]], add=True)
    else:
        cp1 = pltpu.async_copy(g_sh.at[src_all.at[nch - 1]], rows1, sem1)
        pltpu.make_async_copy(g_sh.at[src_all.at[nch - 2]], rows0,
                              sem0).wait()
        pltpu.sync_copy(rows0, acc_sh.at[dst_all.at[nch - 2]], add=True)
        cp1.wait()
        pltpu.sync_copy(rows1, acc_sh.at[dst_all.at[nch - 1]], add=True)


def _sc_mp1(h, e4, zeros_pad):
    np_, f = h.shape
    nch, ch = e4.shape[2:]
    rpt = np_ // _NS
    dpt = rpt // _L

    @functools.partial(
        pl.kernel,
        out_type=(jax.ShapeDtypeStruct((_NC, np_, f), jnp.float32),
                  jax.ShapeDtypeStruct((np_, f), jnp.float32),
                  jax.ShapeDtypeStruct((np_,), jnp.float32)),
        mesh=_mesh(),
        scratch_types=[
            pltpu.VMEM((nch, ch), jnp.int32),
            pltpu.VMEM((nch, ch), jnp.int32),
            pltpu.VMEM((nch, ch), jnp.int32),
            pltpu.VMEM((np_,), jnp.float32),
            pltpu.VMEM((_NS, rpt), jnp.float32),
            pltpu.VMEM((rpt, f), jnp.float32),
            pltpu.VMEM((rpt + _L,), jnp.float32),
            pltpu.VMEM((ch, f), jnp.float32),
            pltpu.VMEM((ch, f), jnp.float32),
            pltpu.VMEM_SHARED((np_, f), jnp.float32),
            pltpu.VMEM_SHARED((np_, f), jnp.float32),
            pltpu.VMEM_SHARED((_NS, np_), jnp.float32),
            pltpu.SemaphoreType.DMA,
            pltpu.SemaphoreType.DMA,
        ],
        compiler_params=_SC_PARAMS,
    )
    def k(h_hbm, e_hbm, z_hbm, s1_hbm, g1_hbm, dinv_hbm,
          src_all, dst_all, dst_oth, hist, parts, hbuf, degv,
          rows0, rows1, g_sh, acc_sh, hist_sh, sem0, sem1):
        c = lax.axis_index("c")
        s = lax.axis_index("s")
        wid = c * _NS + s
        oth = (1 - c) * _NS + s
        r0 = s * rpt

        pltpu.sync_copy(e_hbm.at[0, wid], src_all)
        pltpu.sync_copy(e_hbm.at[1, wid], dst_all)
        pltpu.sync_copy(e_hbm.at[1, oth], dst_oth)
        pltpu.sync_copy(h_hbm.at[pl.ds(r0, rpt)], hbuf)
        pltpu.sync_copy(z_hbm.at[pl.ds(r0, rpt)], acc_sh.at[pl.ds(r0, rpt)])

        zero16 = jnp.zeros((_L,), jnp.float32)

        def zrow(i, carry):
            hist[pl.ds(i * _L, _L)] = zero16
            return carry
        lax.fori_loop(0, np_ // _L, zrow, 0)

        ones = jnp.full((_L,), 1.0, jnp.float32)
        for blk in (dst_all, dst_oth):
            def hbody(i, carry, blk=blk):
                for j in range(ch // _L):
                    idx = blk[i, pl.ds(j * _L, _L)]
                    plsc.addupdate_scatter(hist, [idx], ones)
                return carry
            lax.fori_loop(0, nch, hbody, 0)

        pltpu.sync_copy(hist, hist_sh.at[s])
        plsc.subcore_barrier()
        for t in range(_NS):
            pltpu.sync_copy(hist_sh.at[t, pl.ds(r0, rpt)], parts.at[t])

        def dbody(i, carry):
            acc = parts[0, pl.ds(i * _L, _L)]
            for t in range(1, _NS):
                acc = acc + parts[t, pl.ds(i * _L, _L)]
            degv[pl.ds(i * _L, _L)] = _rsqrt16(acc + 1.0)
            return carry
        lax.fori_loop(0, rpt // _L, dbody, 0)
        pltpu.sync_copy(degv.at[pl.ds(0, rpt)], dinv_hbm.at[pl.ds(r0, rpt)])

        def gbody(r, carry):
            dv = degv[pl.ds(r, _L)]
            hbuf[r, :] = hbuf[r, :] * dv[0]
            return carry
        lax.fori_loop(0, rpt, gbody, 0)
        pltpu.sync_copy(hbuf, g_sh.at[pl.ds(r0, rpt)])
        pltpu.sync_copy(hbuf, g1_hbm.at[pl.ds(r0, rpt)])
        plsc.subcore_barrier()

        _edge_loop(g_sh, acc_sh, src_all, dst_all, rows0, rows1,
                   sem0, sem1, nch)

        plsc.subcore_barrier()
        pltpu.sync_copy(acc_sh.at[pl.ds(r0, rpt)],
                        s1_hbm.at[c, pl.ds(r0, rpt)])

    return k(h, e4, zeros_pad)


def _sc_mp2(s1, g1, dinv, b1, e4, zeros_pad, n_valid):
    np_, f = g1.shape
    nch, ch = e4.shape[2:]
    rpt = np_ // _NS
    dpt = rpt // _L
    pad_rows = np_ - n_valid

    @functools.partial(
        pl.kernel,
        out_type=(jax.ShapeDtypeStruct((_NC, np_, f), jnp.float32),
                  jax.ShapeDtypeStruct((np_, f), jnp.float32)),
        mesh=_mesh(),
        scratch_types=[
            pltpu.VMEM((nch, ch), jnp.int32),
            pltpu.VMEM((nch, ch), jnp.int32),
            pltpu.VMEM((rpt, f), jnp.float32),
            pltpu.VMEM((rpt, f), jnp.float32),
            pltpu.VMEM((rpt, f), jnp.float32),
            pltpu.VMEM((rpt + _L,), jnp.float32),
            pltpu.VMEM((_L,), jnp.float32),
            pltpu.VMEM((ch, f), jnp.float32),
            pltpu.VMEM((ch, f), jnp.float32),
            pltpu.VMEM_SHARED((np_, f), jnp.float32),
            pltpu.VMEM_SHARED((np_, f), jnp.float32),
            pltpu.SemaphoreType.DMA,
            pltpu.SemaphoreType.DMA,
        ],
        compiler_params=_SC_PARAMS,
    )
    def k(s1_hbm, g1_hbm, dinv_hbm, b1_hbm, e_hbm, z_hbm, s2_hbm, g2_hbm,
          src_all, dst_all, abuf, bbuf, gbuf, degv, b1v,
          rows0, rows1, g_sh, acc_sh, sem0, sem1):
        c = lax.axis_index("c")
        s = lax.axis_index("s")
        wid = c * _NS + s
        r0 = s * rpt

        pltpu.sync_copy(e_hbm.at[0, wid], src_all)
        pltpu.sync_copy(e_hbm.at[1, wid], dst_all)
        pltpu.sync_copy(s1_hbm.at[0, pl.ds(r0, rpt)], abuf)
        pltpu.sync_copy(s1_hbm.at[1, pl.ds(r0, rpt)], bbuf)
        pltpu.sync_copy(g1_hbm.at[pl.ds(r0, rpt)], gbuf)
        pltpu.sync_copy(dinv_hbm.at[pl.ds(r0, rpt)], degv.at[pl.ds(0, rpt)])
        pltpu.sync_copy(b1_hbm, b1v)
        pltpu.sync_copy(z_hbm.at[pl.ds(r0, rpt)], acc_sh.at[pl.ds(r0, rpt)])

        b1vec = b1v[...]

        def gbody(r, carry):
            dvv = degv[pl.ds(r, _L)]
            dv = dvv[0]
            m = (abuf[r, :] + bbuf[r, :] + gbuf[r, :]) * dv + b1vec
            abuf[r, :] = jnp.maximum(m, 0.0) * dv
            return carry
        lax.fori_loop(0, rpt, gbody, 0)

        if pad_rows:
            zero16 = jnp.zeros((_L,), jnp.float32)

            @pl.when(s == _NS - 1)
            def _():
                def zbody(r, carry):
                    abuf[r, :] = zero16
                    return carry
                lax.fori_loop(rpt - pad_rows, rpt, zbody, 0)

        pltpu.sync_copy(abuf, g_sh.at[pl.ds(r0, rpt)])
        pltpu.sync_copy(abuf, g2_hbm.at[pl.ds(r0, rpt)])
        plsc.subcore_barrier()

        _edge_loop(g_sh, acc_sh, src_all, dst_all, rows0, rows1,
                   sem0, sem1, nch)

        plsc.subcore_barrier()
        pltpu.sync_copy(acc_sh.at[pl.ds(r0, rpt)],
                        s2_hbm.at[c, pl.ds(r0, rpt)])

    return k(s1, g1, dinv, b1, e4, zeros_pad)


def _tc_h(x, w1, np_):
    n = x.shape[0]
    f = w1.shape[1]

    def body(x_ref, w_ref, o_ref):
        h = jnp.dot(x_ref[...], w_ref[...],
                    preferred_element_type=jnp.float32)
        pad = jnp.zeros((np_ - n, f), jnp.float32)
        o_ref[...] = jnp.concatenate([h, pad], axis=0)

    return pl.pallas_call(
        body,
        out_shape=jax.ShapeDtypeStruct((np_, f), jnp.float32),
    )(x, w1)


def _tc_final(s2, g2, dinv_col, w2, b2, k0, k1, k2, fc_wt, fc_b, n):
    ncls = fc_wt.shape[1]

    def body(s_ref, g_ref, d_ref, w2_ref, b2_ref, k0_ref, k1_ref, k2_ref,
             fw_ref, fb_ref, o_ref):
        m = d_ref[...] * (s_ref[0, :n] + s_ref[1, :n] + g_ref[:n])
        h2 = jnp.dot(m, w2_ref[...],
                     preferred_element_type=jnp.float32) + b2_ref[...]
        a = jnp.dot(h2, k0_ref[...], preferred_element_type=jnp.float32)
        b = jnp.dot(h2, k1_ref[...], preferred_element_type=jnp.float32)
        c = jnp.dot(h2, k2_ref[...], preferred_element_type=jnp.float32)
        row = lax.broadcasted_iota(jnp.int32, a.shape, 0)
        a_dn = jnp.where(row == 0, 0.0, pltpu.roll(a, 1, 0))
        c_up = jnp.where(row == n - 1, 0.0, pltpu.roll(c, n - 1, 0))
        y = b + a_dn + c_up
        p = jnp.max(y, axis=0, keepdims=True)
        o_ref[...] = jnp.dot(p, fw_ref[...],
                             preferred_element_type=jnp.float32) + fb_ref[...]

    return pl.pallas_call(
        body,
        out_shape=jax.ShapeDtypeStruct((1, ncls), jnp.float32),
    )(s2, g2, dinv_col, w2, b2.reshape(1, -1), k0, k1, k2,
      fc_wt, fc_b.reshape(1, -1))


def kernel(x, edge_index, W1, b1, W2, b2, conv_w, conv_b, fc_w, fc_b):
    n = x.shape[0]
    f = W1.shape[1]
    e = edge_index.shape[1]
    ept = e // _NW
    nch = ept // _CH
    assert e % _NW == 0 and ept % _CH == 0
    blk = _NS * 40
    np_ = ((n + blk - 1) // blk) * blk
    e4 = edge_index.reshape(2, _NW, nch, _CH)

    zeros_pad = jnp.zeros((np_, f), jnp.float32)

    h = _tc_h(x, W1, np_)
    s1, g1, dinv_t = _sc_mp1(h, e4, zeros_pad)
    s2, g2 = _sc_mp2(s1, g1, dinv_t, b1, e4, zeros_pad, n)

    dinv_col = dinv_t.reshape(np_, 1)[:n]

    k0 = conv_w[:, :, 0, 0].T
    k1 = conv_w[:, :, 0, 1].T
    k2 = conv_w[:, :, 0, 2].T
    y_bias = conv_b.reshape(1, -1)

    out = _tc_final(s2, g2, dinv_col, W2, b2, k0, k1, k2,
                    fc_w.T, fc_b + y_bias @ fc_w.T, n)
    return out


# speedup vs baseline: 1.4561x; 1.0069x over previous
import functools

import jax
import jax.numpy as jnp
from jax import lax
from jax.experimental import pallas as pl
from jax.experimental.pallas import tpu as pltpu
from jax.experimental.pallas import tpu_sc as plsc

_NC = 2
_NS = 16
_NW = _NC * _NS
_L = 16

_CH = 80

_SC_PARAMS = pltpu.CompilerParams(needs_layout_passes=False,
                                  use_tc_tiling_on_sc=False)


def _mesh():
    return plsc.VectorSubcoreMesh(core_axis_name="c", subcore_axis_name="s")


def _rsqrt16(x):
    y = plsc.bitcast(x, jnp.int32)
    y = jnp.int32(0x5F3759DF) - lax.shift_right_logical(y, 1)
    r = plsc.bitcast(y, jnp.float32)
    for _ in range(3):
        r = r * (1.5 - 0.5 * x * r * r)
    return r


def _edge_loop(g_sh, acc_sh, src_all, dst_all, rows0, rows1, sem0, sem1,
               nch):
    npair = (nch - 1) // 2
    pltpu.async_copy(g_sh.at[src_all.at[0]], rows0, sem0)

    def pair(j, carry):
        b1 = 2 * j + 1
        cp1 = pltpu.async_copy(g_sh.at[src_all.at[b1]], rows1, sem1)
        pltpu.make_async_copy(g_sh.at[src_all.at[b1 - 1]], rows0,
                              sem0).wait()
        pltpu.sync_copy(rows0, acc_sh.at[dst_all.at[b1 - 1]], add=True)
        pltpu.async_copy(g_sh.at[src_all.at[b1 + 1]], rows0, sem0)
        cp1.wait()
        pltpu.sync_copy(rows1, acc_sh.at[dst_all.at[b1]], add=True)
        return carry

    lax.fori_loop(0, npair, pair, 0)
    if nch % 2 == 1:
        pltpu.make_async_copy(g_sh.at[src_all.at[nch - 1]], rows0,
                              sem0).wait()
        pltpu.sync_copy(rows0, acc_sh.at[dst_all.at[nch - 1]], add=True)
    else:
        cp1 = pltpu.async_copy(g_sh.at[src_all.at[nch - 1]], rows1, sem1)
        pltpu.make_async_copy(g_sh.at[src_all.at[nch - 2]], rows0,
                              sem0).wait()
        pltpu.sync_copy(rows0, acc_sh.at[dst_all.at[nch - 2]], add=True)
        cp1.wait()
        pltpu.sync_copy(rows1, acc_sh.at[dst_all.at[nch - 1]], add=True)


def _sc_mp1(h, e4, zeros_pad, zeros_flat):
    np_, f = h.shape
    nch, ch = e4.shape[2:]
    rpt = np_ // _NS
    dpt = rpt // _L

    @functools.partial(
        pl.kernel,
        out_type=(jax.ShapeDtypeStruct((_NC, np_, f), jnp.float32),
                  jax.ShapeDtypeStruct((np_, f), jnp.float32),
                  jax.ShapeDtypeStruct((np_,), jnp.float32)),
        mesh=_mesh(),
        scratch_types=[
            pltpu.VMEM((nch, ch), jnp.int32),
            pltpu.VMEM((nch, ch), jnp.int32),
            pltpu.VMEM((nch, ch), jnp.int32),
            pltpu.VMEM((np_,), jnp.float32),
            pltpu.VMEM((_NS, rpt), jnp.float32),
            pltpu.VMEM((rpt, f), jnp.float32),
            pltpu.VMEM((rpt + _L,), jnp.float32),
            pltpu.VMEM((ch, f), jnp.float32),
            pltpu.VMEM((ch, f), jnp.float32),
            pltpu.VMEM_SHARED((np_, f), jnp.float32),
            pltpu.VMEM_SHARED((np_, f), jnp.float32),
            pltpu.VMEM_SHARED((_NS, np_), jnp.float32),
            pltpu.SemaphoreType.DMA,
            pltpu.SemaphoreType.DMA,
        ],
        compiler_params=_SC_PARAMS,
    )
    def k(h_hbm, e_hbm, z_hbm, z1_hbm, s1_hbm, g1_hbm, dinv_hbm,
          src_all, dst_all, dst_oth, hist, parts, hbuf, degv,
          rows0, rows1, g_sh, acc_sh, hist_sh, sem0, sem1):
        c = lax.axis_index("c")
        s = lax.axis_index("s")
        wid = c * _NS + s
        oth = (1 - c) * _NS + s
        r0 = s * rpt

        pltpu.sync_copy(e_hbm.at[0, wid], src_all)
        pltpu.sync_copy(e_hbm.at[1, wid], dst_all)
        pltpu.sync_copy(e_hbm.at[1, oth], dst_oth)
        pltpu.sync_copy(h_hbm.at[pl.ds(r0, rpt)], hbuf)
        pltpu.sync_copy(z_hbm.at[pl.ds(r0, rpt)], acc_sh.at[pl.ds(r0, rpt)])
        pltpu.sync_copy(z1_hbm, hist)

        ones = jnp.full((_L,), 1.0, jnp.float32)
        for blk in (dst_all, dst_oth):
            def hbody(i, carry, blk=blk):
                for j in range(ch // _L):
                    idx = blk[i, pl.ds(j * _L, _L)]
                    plsc.addupdate_scatter(hist, [idx], ones)
                return carry
            lax.fori_loop(0, nch, hbody, 0)

        pltpu.sync_copy(hist, hist_sh.at[s])
        plsc.subcore_barrier()
        for t in range(_NS):
            pltpu.sync_copy(hist_sh.at[t, pl.ds(r0, rpt)], parts.at[t])

        def dbody(i, carry):
            acc = parts[0, pl.ds(i * _L, _L)]
            for t in range(1, _NS):
                acc = acc + parts[t, pl.ds(i * _L, _L)]
            degv[pl.ds(i * _L, _L)] = _rsqrt16(acc + 1.0)
            return carry
        lax.fori_loop(0, rpt // _L, dbody, 0)
        pltpu.sync_copy(degv.at[pl.ds(0, rpt)], dinv_hbm.at[pl.ds(r0, rpt)])

        def gbody(q, carry):
            for u in range(4):
                r = q * 4 + u
                dv = degv[pl.ds(r, _L)]
                hbuf[r, :] = hbuf[r, :] * dv[0]
            return carry
        lax.fori_loop(0, rpt // 4, gbody, 0)
        pltpu.sync_copy(hbuf, g_sh.at[pl.ds(r0, rpt)])
        pltpu.sync_copy(hbuf, g1_hbm.at[pl.ds(r0, rpt)])
        plsc.subcore_barrier()

        _edge_loop(g_sh, acc_sh, src_all, dst_all, rows0, rows1,
                   sem0, sem1, nch)

        plsc.subcore_barrier()
        pltpu.sync_copy(acc_sh.at[pl.ds(r0, rpt)],
                        s1_hbm.at[c, pl.ds(r0, rpt)])

    return k(h, e4, zeros_pad, zeros_flat)


def _sc_mp2(s1, g1, dinv, b1, e4, zeros_pad, n_valid):
    np_, f = g1.shape
    nch, ch = e4.shape[2:]
    rpt = np_ // _NS
    dpt = rpt // _L
    pad_rows = np_ - n_valid

    @functools.partial(
        pl.kernel,
        out_type=(jax.ShapeDtypeStruct((_NC, np_, f), jnp.float32),
                  jax.ShapeDtypeStruct((np_, f), jnp.float32)),
        mesh=_mesh(),
        scratch_types=[
            pltpu.VMEM((nch, ch), jnp.int32),
            pltpu.VMEM((nch, ch), jnp.int32),
            pltpu.VMEM((rpt, f), jnp.float32),
            pltpu.VMEM((rpt, f), jnp.float32),
            pltpu.VMEM((rpt, f), jnp.float32),
            pltpu.VMEM((rpt + _L,), jnp.float32),
            pltpu.VMEM((_L,), jnp.float32),
            pltpu.VMEM((ch, f), jnp.float32),
            pltpu.VMEM((ch, f), jnp.float32),
            pltpu.VMEM_SHARED((np_, f), jnp.float32),
            pltpu.VMEM_SHARED((np_, f), jnp.float32),
            pltpu.SemaphoreType.DMA,
            pltpu.SemaphoreType.DMA,
        ],
        compiler_params=_SC_PARAMS,
    )
    def k(s1_hbm, g1_hbm, dinv_hbm, b1_hbm, e_hbm, z_hbm, s2_hbm, g2_hbm,
          src_all, dst_all, abuf, bbuf, gbuf, degv, b1v,
          rows0, rows1, g_sh, acc_sh, sem0, sem1):
        c = lax.axis_index("c")
        s = lax.axis_index("s")
        wid = c * _NS + s
        r0 = s * rpt

        pltpu.sync_copy(e_hbm.at[0, wid], src_all)
        pltpu.sync_copy(e_hbm.at[1, wid], dst_all)
        pltpu.sync_copy(s1_hbm.at[0, pl.ds(r0, rpt)], abuf)
        pltpu.sync_copy(s1_hbm.at[1, pl.ds(r0, rpt)], bbuf)
        pltpu.sync_copy(g1_hbm.at[pl.ds(r0, rpt)], gbuf)
        pltpu.sync_copy(dinv_hbm.at[pl.ds(r0, rpt)], degv.at[pl.ds(0, rpt)])
        pltpu.sync_copy(b1_hbm, b1v)
        pltpu.sync_copy(z_hbm.at[pl.ds(r0, rpt)], acc_sh.at[pl.ds(r0, rpt)])

        b1vec = b1v[...]

        def gbody(q, carry):
            for u in range(4):
                r = q * 4 + u
                dv = degv[pl.ds(r, _L)][0]
                m = (abuf[r, :] + bbuf[r, :] + gbuf[r, :]) * dv + b1vec
                abuf[r, :] = jnp.maximum(m, 0.0) * dv
            return carry
        lax.fori_loop(0, rpt // 4, gbody, 0)

        if pad_rows:
            zero16 = jnp.zeros((_L,), jnp.float32)

            @pl.when(s == _NS - 1)
            def _():
                def zbody(r, carry):
                    abuf[r, :] = zero16
                    return carry
                lax.fori_loop(rpt - pad_rows, rpt, zbody, 0)

        pltpu.sync_copy(abuf, g_sh.at[pl.ds(r0, rpt)])
        pltpu.sync_copy(abuf, g2_hbm.at[pl.ds(r0, rpt)])
        plsc.subcore_barrier()

        _edge_loop(g_sh, acc_sh, src_all, dst_all, rows0, rows1,
                   sem0, sem1, nch)

        plsc.subcore_barrier()
        pltpu.sync_copy(acc_sh.at[pl.ds(r0, rpt)],
                        s2_hbm.at[c, pl.ds(r0, rpt)])

    return k(s1, g1, dinv, b1, e4, zeros_pad)


def _tc_h(x, w1, np_):
    n = x.shape[0]
    f = w1.shape[1]

    def body(x_ref, w_ref, o_ref):
        h = jnp.dot(x_ref[...], w_ref[...],
                    preferred_element_type=jnp.float32)
        pad = jnp.zeros((np_ - n, f), jnp.float32)
        o_ref[...] = jnp.concatenate([h, pad], axis=0)

    return pl.pallas_call(
        body,
        out_shape=jax.ShapeDtypeStruct((np_, f), jnp.float32),
    )(x, w1)


def _tc_final(s2, g2, dinv_col, w2, b2, k0, k1, k2, fc_wt, fc_b, n):
    ncls = fc_wt.shape[1]

    def body(s_ref, g_ref, d_ref, w2_ref, b2_ref, k0_ref, k1_ref, k2_ref,
             fw_ref, fb_ref, o_ref):
        m = d_ref[...] * (s_ref[0, :n] + s_ref[1, :n] + g_ref[:n])
        h2 = jnp.dot(m, w2_ref[...],
                     preferred_element_type=jnp.float32) + b2_ref[...]
        a = jnp.dot(h2, k0_ref[...], preferred_element_type=jnp.float32)
        b = jnp.dot(h2, k1_ref[...], preferred_element_type=jnp.float32)
        c = jnp.dot(h2, k2_ref[...], preferred_element_type=jnp.float32)
        row = lax.broadcasted_iota(jnp.int32, a.shape, 0)
        a_dn = jnp.where(row == 0, 0.0, pltpu.roll(a, 1, 0))
        c_up = jnp.where(row == n - 1, 0.0, pltpu.roll(c, n - 1, 0))
        y = b + a_dn + c_up
        p = jnp.max(y, axis=0, keepdims=True)
        o_ref[...] = jnp.dot(p, fw_ref[...],
                             preferred_element_type=jnp.float32) + fb_ref[...]

    return pl.pallas_call(
        body,
        out_shape=jax.ShapeDtypeStruct((1, ncls), jnp.float32),
    )(s2, g2, dinv_col, w2, b2.reshape(1, -1), k0, k1, k2,
      fc_wt, fc_b.reshape(1, -1))


def kernel(x, edge_index, W1, b1, W2, b2, conv_w, conv_b, fc_w, fc_b):
    n = x.shape[0]
    f = W1.shape[1]
    e = edge_index.shape[1]
    ept = e // _NW
    nch = ept // _CH
    assert e % _NW == 0 and ept % _CH == 0
    blk = _NS * 40
    np_ = ((n + blk - 1) // blk) * blk
    e4 = edge_index.reshape(2, _NW, nch, _CH)

    zeros_pad = jnp.zeros((np_, f), jnp.float32)
    zeros_flat = jnp.zeros((np_,), jnp.float32)

    h = _tc_h(x, W1, np_)
    s1, g1, dinv_t = _sc_mp1(h, e4, zeros_pad, zeros_flat)
    s2, g2 = _sc_mp2(s1, g1, dinv_t, b1, e4, zeros_pad, n)

    dinv_col = dinv_t.reshape(np_, 1)[:n]

    k0 = conv_w[:, :, 0, 0].T
    k1 = conv_w[:, :, 0, 1].T
    k2 = conv_w[:, :, 0, 2].T
    y_bias = conv_b.reshape(1, -1)

    out = _tc_final(s2, g2, dinv_col, W2, b2, k0, k1, k2,
                    fc_w.T, fc_b + y_bias @ fc_w.T, n)
    return out


# speedup vs baseline: 1.5085x; 1.0360x over previous
import functools

import jax
import jax.numpy as jnp
from jax import lax
from jax.experimental import pallas as pl
from jax.experimental.pallas import tpu as pltpu
from jax.experimental.pallas import tpu_sc as plsc

_NC = 2
_NS = 16
_NW = _NC * _NS
_L = 16

_CH = 80

_SC_PARAMS = pltpu.CompilerParams(needs_layout_passes=False,
                                  use_tc_tiling_on_sc=False)


def _mesh():
    return plsc.VectorSubcoreMesh(core_axis_name="c", subcore_axis_name="s")


def _rsqrt16(x):
    y = plsc.bitcast(x, jnp.int32)
    y = jnp.int32(0x5F3759DF) - lax.shift_right_logical(y, 1)
    r = plsc.bitcast(y, jnp.float32)
    for _ in range(3):
        r = r * (1.5 - 0.5 * x * r * r)
    return r


def _edge_loop(g_sh, acc_sh, src_all, dst_all, rows, gsem, ssem, nch):
    assert nch % 4 == 1 and nch >= 9

    def gather(c, b):
        return pltpu.async_copy(g_sh.at[src_all.at[c]], rows[b], gsem[b])

    def scat(c, b):
        return pltpu.async_copy(rows[b], acc_sh.at[dst_all.at[c]],
                                ssem[b], add=True)

    def gwait(c, b):
        pltpu.make_async_copy(g_sh.at[src_all.at[c]], rows[b],
                              gsem[b]).wait()

    def swait(c, b):
        pltpu.make_async_copy(rows[b], acc_sh.at[dst_all.at[c]],
                              ssem[b]).wait()

    for b in range(4):
        gather(b, b)

    nquad = (nch - 5) // 4

    def quad(j, carry):
        c0 = 4 * j
        for b in range(4):
            gwait(c0 + b, b)
            scat(c0 + b, b)
        for b in range(4):
            swait(c0 + b, b)
            gather(c0 + 4 + b, b)
        return carry

    lax.fori_loop(0, nquad, quad, 0)
    c0 = 4 * nquad
    for b in range(4):
        gwait(c0 + b, b)
        scat(c0 + b, b)
    swait(c0, 0)
    gather(nch - 1, 0)
    gwait(nch - 1, 0)
    pltpu.sync_copy(rows[0], acc_sh.at[dst_all.at[nch - 1]], add=True)
    for b in range(1, 4):
        swait(c0 + b, b)


def _sc_mp1(h, e4, zeros_pad, zeros_flat):
    np_, f = h.shape
    nch, ch = e4.shape[2:]
    rpt = np_ // _NS
    dpt = rpt // _L

    @functools.partial(
        pl.kernel,
        out_type=(jax.ShapeDtypeStruct((_NC, np_, f), jnp.float32),
                  jax.ShapeDtypeStruct((np_, f), jnp.float32),
                  jax.ShapeDtypeStruct((np_,), jnp.float32)),
        mesh=_mesh(),
        scratch_types=[
            pltpu.VMEM((nch, ch), jnp.int32),
            pltpu.VMEM((nch, ch), jnp.int32),
            pltpu.VMEM((nch, ch), jnp.int32),
            pltpu.VMEM((np_,), jnp.float32),
            pltpu.VMEM((_NS, rpt), jnp.float32),
            pltpu.VMEM((rpt, f), jnp.float32),
            pltpu.VMEM((rpt + _L,), jnp.float32),
            [pltpu.VMEM((ch, f), jnp.float32) for _ in range(4)],
            pltpu.VMEM_SHARED((np_, f), jnp.float32),
            pltpu.VMEM_SHARED((np_, f), jnp.float32),
            pltpu.VMEM_SHARED((_NS, np_), jnp.float32),
            [pltpu.SemaphoreType.DMA for _ in range(4)],
            [pltpu.SemaphoreType.DMA for _ in range(4)],
        ],
        compiler_params=_SC_PARAMS,
    )
    def k(h_hbm, e_hbm, z_hbm, z1_hbm, s1_hbm, g1_hbm, dinv_hbm,
          src_all, dst_all, dst_oth, hist, parts, hbuf, degv,
          rows, g_sh, acc_sh, hist_sh, gsem, ssem):
        c = lax.axis_index("c")
        s = lax.axis_index("s")
        wid = c * _NS + s
        oth = (1 - c) * _NS + s
        r0 = s * rpt

        pltpu.sync_copy(e_hbm.at[0, wid], src_all)
        pltpu.sync_copy(e_hbm.at[1, wid], dst_all)
        pltpu.sync_copy(e_hbm.at[1, oth], dst_oth)
        pltpu.sync_copy(h_hbm.at[pl.ds(r0, rpt)], hbuf)
        pltpu.sync_copy(z_hbm.at[pl.ds(r0, rpt)], acc_sh.at[pl.ds(r0, rpt)])
        pltpu.sync_copy(z1_hbm, hist)

        ones = jnp.full((_L,), 1.0, jnp.float32)
        for blk in (dst_all, dst_oth):
            def hbody(i, carry, blk=blk):
                for j in range(ch // _L):
                    idx = blk[i, pl.ds(j * _L, _L)]
                    plsc.addupdate_scatter(hist, [idx], ones)
                return carry
            lax.fori_loop(0, nch, hbody, 0)

        pltpu.sync_copy(hist, hist_sh.at[s])
        plsc.subcore_barrier()
        for t in range(_NS):
            pltpu.sync_copy(hist_sh.at[t, pl.ds(r0, rpt)], parts.at[t])

        def dbody(i, carry):
            acc = parts[0, pl.ds(i * _L, _L)]
            for t in range(1, _NS):
                acc = acc + parts[t, pl.ds(i * _L, _L)]
            degv[pl.ds(i * _L, _L)] = _rsqrt16(acc + 1.0)
            return carry
        lax.fori_loop(0, rpt // _L, dbody, 0)
        pltpu.sync_copy(degv.at[pl.ds(0, rpt)], dinv_hbm.at[pl.ds(r0, rpt)])

        def gbody(q, carry):
            for u in range(4):
                r = q * 4 + u
                dv = degv[pl.ds(r, _L)]
                hbuf[r, :] = hbuf[r, :] * dv[0]
            return carry
        lax.fori_loop(0, rpt // 4, gbody, 0)
        pltpu.sync_copy(hbuf, g_sh.at[pl.ds(r0, rpt)])
        pltpu.sync_copy(hbuf, g1_hbm.at[pl.ds(r0, rpt)])
        plsc.subcore_barrier()

        _edge_loop(g_sh, acc_sh, src_all, dst_all, rows, gsem, ssem, nch)

        plsc.subcore_barrier()
        pltpu.sync_copy(acc_sh.at[pl.ds(r0, rpt)],
                        s1_hbm.at[c, pl.ds(r0, rpt)])

    return k(h, e4, zeros_pad, zeros_flat)


def _sc_mp2(s1, g1, dinv, b1, e4, zeros_pad, n_valid):
    np_, f = g1.shape
    nch, ch = e4.shape[2:]
    rpt = np_ // _NS
    dpt = rpt // _L
    pad_rows = np_ - n_valid

    @functools.partial(
        pl.kernel,
        out_type=(jax.ShapeDtypeStruct((_NC, np_, f), jnp.float32),
                  jax.ShapeDtypeStruct((np_, f), jnp.float32)),
        mesh=_mesh(),
        scratch_types=[
            pltpu.VMEM((nch, ch), jnp.int32),
            pltpu.VMEM((nch, ch), jnp.int32),
            pltpu.VMEM((rpt, f), jnp.float32),
            pltpu.VMEM((rpt, f), jnp.float32),
            pltpu.VMEM((rpt, f), jnp.float32),
            pltpu.VMEM((rpt + _L,), jnp.float32),
            pltpu.VMEM((_L,), jnp.float32),
            [pltpu.VMEM((ch, f), jnp.float32) for _ in range(4)],
            pltpu.VMEM_SHARED((np_, f), jnp.float32),
            pltpu.VMEM_SHARED((np_, f), jnp.float32),
            [pltpu.SemaphoreType.DMA for _ in range(4)],
            [pltpu.SemaphoreType.DMA for _ in range(4)],
        ],
        compiler_params=_SC_PARAMS,
    )
    def k(s1_hbm, g1_hbm, dinv_hbm, b1_hbm, e_hbm, z_hbm, s2_hbm, g2_hbm,
          src_all, dst_all, abuf, bbuf, gbuf, degv, b1v,
          rows, g_sh, acc_sh, gsem, ssem):
        c = lax.axis_index("c")
        s = lax.axis_index("s")
        wid = c * _NS + s
        r0 = s * rpt

        pltpu.sync_copy(e_hbm.at[0, wid], src_all)
        pltpu.sync_copy(e_hbm.at[1, wid], dst_all)
        pltpu.sync_copy(s1_hbm.at[0, pl.ds(r0, rpt)], abuf)
        pltpu.sync_copy(s1_hbm.at[1, pl.ds(r0, rpt)], bbuf)
        pltpu.sync_copy(g1_hbm.at[pl.ds(r0, rpt)], gbuf)
        pltpu.sync_copy(dinv_hbm.at[pl.ds(r0, rpt)], degv.at[pl.ds(0, rpt)])
        pltpu.sync_copy(b1_hbm, b1v)
        pltpu.sync_copy(z_hbm.at[pl.ds(r0, rpt)], acc_sh.at[pl.ds(r0, rpt)])

        b1vec = b1v[...]

        def gbody(q, carry):
            for u in range(4):
                r = q * 4 + u
                dv = degv[pl.ds(r, _L)][0]
                m = (abuf[r, :] + bbuf[r, :] + gbuf[r, :]) * dv + b1vec
                abuf[r, :] = jnp.maximum(m, 0.0) * dv
            return carry
        lax.fori_loop(0, rpt // 4, gbody, 0)

        if pad_rows:
            zero16 = jnp.zeros((_L,), jnp.float32)

            @pl.when(s == _NS - 1)
            def _():
                def zbody(r, carry):
                    abuf[r, :] = zero16
                    return carry
                lax.fori_loop(rpt - pad_rows, rpt, zbody, 0)

        pltpu.sync_copy(abuf, g_sh.at[pl.ds(r0, rpt)])
        pltpu.sync_copy(abuf, g2_hbm.at[pl.ds(r0, rpt)])
        plsc.subcore_barrier()

        _edge_loop(g_sh, acc_sh, src_all, dst_all, rows, gsem, ssem, nch)

        plsc.subcore_barrier()
        pltpu.sync_copy(acc_sh.at[pl.ds(r0, rpt)],
                        s2_hbm.at[c, pl.ds(r0, rpt)])

    return k(s1, g1, dinv, b1, e4, zeros_pad)


def _tc_h(x, w1, np_):
    n = x.shape[0]
    f = w1.shape[1]

    def body(x_ref, w_ref, o_ref):
        h = jnp.dot(x_ref[...], w_ref[...],
                    preferred_element_type=jnp.float32)
        pad = jnp.zeros((np_ - n, f), jnp.float32)
        o_ref[...] = jnp.concatenate([h, pad], axis=0)

    return pl.pallas_call(
        body,
        out_shape=jax.ShapeDtypeStruct((np_, f), jnp.float32),
    )(x, w1)


def _tc_final(s2, g2, dinv_col, w2, b2, k0, k1, k2, fc_wt, fc_b, n):
    ncls = fc_wt.shape[1]

    def body(s_ref, g_ref, d_ref, w2_ref, b2_ref, k0_ref, k1_ref, k2_ref,
             fw_ref, fb_ref, o_ref):
        m = d_ref[...] * (s_ref[0, :n] + s_ref[1, :n] + g_ref[:n])
        h2 = jnp.dot(m, w2_ref[...],
                     preferred_element_type=jnp.float32) + b2_ref[...]
        a = jnp.dot(h2, k0_ref[...], preferred_element_type=jnp.float32)
        b = jnp.dot(h2, k1_ref[...], preferred_element_type=jnp.float32)
        c = jnp.dot(h2, k2_ref[...], preferred_element_type=jnp.float32)
        row = lax.broadcasted_iota(jnp.int32, a.shape, 0)
        a_dn = jnp.where(row == 0, 0.0, pltpu.roll(a, 1, 0))
        c_up = jnp.where(row == n - 1, 0.0, pltpu.roll(c, n - 1, 0))
        y = b + a_dn + c_up
        p = jnp.max(y, axis=0, keepdims=True)
        o_ref[...] = jnp.dot(p, fw_ref[...],
                             preferred_element_type=jnp.float32) + fb_ref[...]

    return pl.pallas_call(
        body,
        out_shape=jax.ShapeDtypeStruct((1, ncls), jnp.float32),
    )(s2, g2, dinv_col, w2, b2.reshape(1, -1), k0, k1, k2,
      fc_wt, fc_b.reshape(1, -1))


def kernel(x, edge_index, W1, b1, W2, b2, conv_w, conv_b, fc_w, fc_b):
    n = x.shape[0]
    f = W1.shape[1]
    e = edge_index.shape[1]
    ept = e // _NW
    nch = ept // _CH
    assert e % _NW == 0 and ept % _CH == 0
    blk = _NS * 40
    np_ = ((n + blk - 1) // blk) * blk
    e4 = edge_index.reshape(2, _NW, nch, _CH)

    zeros_pad = jnp.zeros((np_, f), jnp.float32)
    zeros_flat = jnp.zeros((np_,), jnp.float32)

    h = _tc_h(x, W1, np_)
    s1, g1, dinv_t = _sc_mp1(h, e4, zeros_pad, zeros_flat)
    s2, g2 = _sc_mp2(s1, g1, dinv_t, b1, e4, zeros_pad, n)

    dinv_col = dinv_t.reshape(np_, 1)[:n]

    k0 = conv_w[:, :, 0, 0].T
    k1 = conv_w[:, :, 0, 1].T
    k2 = conv_w[:, :, 0, 2].T
    y_bias = conv_b.reshape(1, -1)

    out = _tc_final(s2, g2, dinv_col, W2, b2, k0, k1, k2,
                    fc_w.T, fc_b + y_bias @ fc_w.T, n)
    return out
